# Initial kernel scaffold; baseline (speedup 1.0000x reference)
import jax
import jax.numpy as jnp
from jax.experimental import pallas as pl


def kernel(ent_em, rel_em, adj1_index, adj2_rows, adj2_cols, params):
    raise NotImplementedError("write your pallas kernel here")


# speedup vs baseline: 17.4203x; 17.4203x over previous
import functools

import jax
import jax.numpy as jnp
from jax import lax
from jax.experimental import pallas as pl
from jax.experimental.pallas import tpu as pltpu
from jax.experimental.pallas import tpu_sc as plsc

N = 10000
R = 1000
D = 128
E1 = 320000
E2 = 160000

NC, NS, L = 2, 16, 16
NW = NC * NS
K = 80
NP = 10240
RP = 1008
CH1 = E1 // (NW * K)
E2P = 161280
CH2 = E2P // (NW * K)
RB = 1024

_EPS = 1e-6


def _ln(x, g, b):
    m = jnp.mean(x, axis=1, keepdims=True)
    v = jnp.mean((x - m) ** 2, axis=1, keepdims=True)
    return (x - m) / jnp.sqrt(v + _EPS) * g + b


def _dense1_body(x_ref, wk_ref, wb_ref, w1_ref, b1_ref, w2_ref, b2_ref,
                 g_ref, b_ref, val_ref, s1_ref, s2_ref):
    x = x_ref[...]
    q = _ln(x, g_ref[...], b_ref[...])
    val_ref[...] = jnp.dot(q, wk_ref[...],
                           preferred_element_type=jnp.float32) + wb_ref[...]
    q1 = jnp.dot(q, w1_ref[...], preferred_element_type=jnp.float32) + b1_ref[...]
    q2 = jnp.dot(q, w2_ref[...], preferred_element_type=jnp.float32) + b2_ref[...]
    s1_ref[...] = jnp.tanh(jnp.sum(q1, axis=1))
    s2_ref[...] = jnp.tanh(jnp.sum(q2, axis=1))


def _dense1(x, wk, wb, w1, b1, w2, b2, g, b):
    return pl.pallas_call(
        _dense1_body,
        grid=(NP // RB,),
        in_specs=[
            pl.BlockSpec((RB, D), lambda i: (i, 0)),
            pl.BlockSpec((D, D), lambda i: (0, 0)),
            pl.BlockSpec((1, D), lambda i: (0, 0)),
            pl.BlockSpec((D, D), lambda i: (0, 0)),
            pl.BlockSpec((1, D), lambda i: (0, 0)),
            pl.BlockSpec((D, D), lambda i: (0, 0)),
            pl.BlockSpec((1, D), lambda i: (0, 0)),
            pl.BlockSpec((1, D), lambda i: (0, 0)),
            pl.BlockSpec((1, D), lambda i: (0, 0)),
        ],
        out_specs=[
            pl.BlockSpec((RB, D), lambda i: (i, 0)),
            pl.BlockSpec((RB,), lambda i: (i,)),
            pl.BlockSpec((RB,), lambda i: (i,)),
        ],
        out_shape=[
            jax.ShapeDtypeStruct((NP, D), jnp.float32),
            jax.ShapeDtypeStruct((NP,), jnp.float32),
            jax.ShapeDtypeStruct((NP,), jnp.float32),
        ],
    )(x, wk, wb, w1, b1, w2, b2, g, b)


def _combine(parts, s_parts):
    num = parts[0] + parts[1]
    den = jnp.sum(s_parts, axis=0)[:, None]
    pos = den > 0.0
    return jnp.where(pos, num / jnp.where(pos, den, 1.0), 0.0)


def _dense2_body(p_ref, ps_ref, w1_ref, b1_ref, g_ref, b_ref, e_ref, s1_ref):
    out1 = _combine(p_ref[...], ps_ref[...])
    e = _ln(out1, g_ref[...], b_ref[...])
    e_ref[...] = e
    qw = jnp.dot(e, w1_ref[...], preferred_element_type=jnp.float32) + b1_ref[...]
    s1_ref[...] = jnp.tanh(jnp.sum(qw * e, axis=1))


def _dense2(parts, s_parts, w1, b1, g, b):
    return pl.pallas_call(
        _dense2_body,
        grid=(NP // RB,),
        in_specs=[
            pl.BlockSpec((2, RB, D), lambda i: (0, i, 0)),
            pl.BlockSpec((NW, RB), lambda i: (0, i)),
            pl.BlockSpec((D, D), lambda i: (0, 0)),
            pl.BlockSpec((1, D), lambda i: (0, 0)),
            pl.BlockSpec((1, D), lambda i: (0, 0)),
            pl.BlockSpec((1, D), lambda i: (0, 0)),
        ],
        out_specs=[
            pl.BlockSpec((RB, D), lambda i: (i, 0)),
            pl.BlockSpec((RB,), lambda i: (i,)),
        ],
        out_shape=[
            jax.ShapeDtypeStruct((NP, D), jnp.float32),
            jax.ShapeDtypeStruct((NP,), jnp.float32),
        ],
    )(parts, s_parts, w1, b1, g, b)


def _dense_rel_body(x_ref, wk_ref, wb_ref, w2_ref, b2_ref, g_ref, b_ref,
                    d_ref, s2_ref):
    x = x_ref[...]
    inp = _ln(x, g_ref[...], b_ref[...])
    d = jnp.dot(inp, wk_ref[...], preferred_element_type=jnp.float32) + wb_ref[...]
    rowid = lax.broadcasted_iota(jnp.int32, (RP, 1), 0)
    d_ref[...] = jnp.where(rowid < R, d, 0.0)
    qw = jnp.dot(inp, w2_ref[...], preferred_element_type=jnp.float32) + b2_ref[...]
    s2_ref[...] = jnp.tanh(jnp.sum(qw * inp, axis=1))


def _dense_rel(x, wk, wb, w2, b2, g, b):
    return pl.pallas_call(
        _dense_rel_body,
        grid=(1,),
        in_specs=[
            pl.BlockSpec((RP, D), lambda i: (0, 0)),
            pl.BlockSpec((D, D), lambda i: (0, 0)),
            pl.BlockSpec((1, D), lambda i: (0, 0)),
            pl.BlockSpec((D, D), lambda i: (0, 0)),
            pl.BlockSpec((1, D), lambda i: (0, 0)),
            pl.BlockSpec((1, D), lambda i: (0, 0)),
            pl.BlockSpec((1, D), lambda i: (0, 0)),
        ],
        out_specs=[
            pl.BlockSpec((RP, D), lambda i: (0, 0)),
            pl.BlockSpec((RP,), lambda i: (0,)),
        ],
        out_shape=[
            jax.ShapeDtypeStruct((RP, D), jnp.float32),
            jax.ShapeDtypeStruct((RP,), jnp.float32),
        ],
    )(x, wk, wb, w2, b2, g, b)


def _dense3_body(p_ref, ps_ref, e_ref, g_ref, b_ref, dk_ref, db_ref, res_ref):
    out2 = _combine(p_ref[...], ps_ref[...])
    er = _ln(out2, g_ref[...], b_ref[...])
    res_ref[...] = e_ref[...] + 0.6 * (
        jnp.dot(er, dk_ref[...], preferred_element_type=jnp.float32) + db_ref[...])


def _dense3(parts, s_parts, e_attn, g, b, dk, db):
    return pl.pallas_call(
        _dense3_body,
        grid=(NP // RB,),
        in_specs=[
            pl.BlockSpec((2, RB, D), lambda i: (0, i, 0)),
            pl.BlockSpec((NW, RB), lambda i: (0, i)),
            pl.BlockSpec((RB, D), lambda i: (i, 0)),
            pl.BlockSpec((1, D), lambda i: (0, 0)),
            pl.BlockSpec((1, D), lambda i: (0, 0)),
            pl.BlockSpec((D, D), lambda i: (0, 0)),
            pl.BlockSpec((1, D), lambda i: (0, 0)),
        ],
        out_specs=pl.BlockSpec((RB, D), lambda i: (i, 0)),
        out_shape=jax.ShapeDtypeStruct((NP, D), jnp.float32),
    )(parts, s_parts, e_attn, g, b, dk, db)


def _mesh():
    return plsc.VectorSubcoreMesh(core_axis_name="c", subcore_axis_name="s",
                                  num_cores=NC, num_subcores=NS)


@functools.lru_cache(maxsize=None)
def _make_weights(tbl_rows, nchunks):
    @functools.partial(
        pl.kernel,
        out_type=[
            jax.ShapeDtypeStruct((NW, nchunks, K), jnp.float32),
            jax.ShapeDtypeStruct((NW, NP), jnp.float32),
        ],
        mesh=_mesh(),
        scratch_types=[
            pltpu.VMEM((nchunks, K), jnp.int32),
            pltpu.VMEM((nchunks, K), jnp.int32),
            pltpu.VMEM((NP,), jnp.float32),
            pltpu.VMEM((tbl_rows,), jnp.float32),
            pltpu.VMEM((nchunks, K), jnp.float32),
            pltpu.VMEM((NP,), jnp.float32),
        ],
        compiler_params=pltpu.CompilerParams(needs_layout_passes=False),
    )
    def weights_kernel(er_hbm, ec_hbm, s1_hbm, s2_hbm, w_hbm, outs_hbm,
                       rv, cv, s1v, s2v, wv, sv):
        cid = lax.axis_index("c")
        sid = lax.axis_index("s")
        wid = sid * NC + cid

        pltpu.sync_copy(er_hbm.at[wid], rv)
        pltpu.sync_copy(ec_hbm.at[wid], cv)
        pltpu.sync_copy(s1_hbm, s1v)
        pltpu.sync_copy(s2_hbm, s2v)

        @pl.loop(0, NP // L)
        def _zero_s(i):
            sv[pl.ds(i * L, L)] = jnp.zeros((L,), jnp.float32)

        lane = lax.iota(jnp.int32, L)
        dummy = jnp.int32(NP - 8)

        @pl.loop(0, nchunks)
        def _chunk(c):
            for i in range(K // L):
                r16 = rv[c, pl.ds(i * L, L)]
                c16 = cv[c, pl.ds(i * L, L)]
                ev = plsc.load_gather(s1v, [r16]) + plsc.load_gather(s2v, [c16])
                ev = jnp.maximum(ev, 0.2 * ev)
                w16 = jnp.exp(ev)
                wv[c, pl.ds(i * L, L)] = w16
                rs, wsr = plsc.sort_key_val(r16, w16)
                prev = rs.at[jnp.maximum(lane - 1, 0)].get(
                    mode="promise_in_bounds")
                nxt = rs.at[jnp.minimum(lane + 1, L - 1)].get(
                    mode="promise_in_bounds")
                first = (rs != prev) | (lane == 0)
                last = (rs != nxt) | (lane == L - 1)
                prefix = plsc.cumsum(wsr)
                start = plsc.cummax(jnp.where(first, lane, 0))
                pb = prefix.at[jnp.maximum(start - 1, 0)].get(
                    mode="promise_in_bounds")
                tot = prefix - jnp.where(start > 0, pb, 0.0)
                sidx = jnp.where(last, rs, dummy)
                plsc.addupdate_scatter(sv, [sidx], tot)

        pltpu.sync_copy(wv, w_hbm.at[wid])
        pltpu.sync_copy(sv, outs_hbm.at[wid])

    return weights_kernel


@functools.lru_cache(maxsize=None)
def _make_numer(tbl_rows, nchunks):
    rows_pt = NP // NS

    @functools.partial(
        pl.kernel,
        out_type=jax.ShapeDtypeStruct((NC, NP, D), jnp.float32),
        mesh=_mesh(),
        scratch_types=[
            pltpu.VMEM((nchunks, K), jnp.int32),
            pltpu.VMEM((K,), jnp.int32),
            pltpu.VMEM((K,), jnp.float32),
            pltpu.VMEM((K, D), jnp.float32),
            pltpu.VMEM_SHARED((NP, D), jnp.float32),
            pltpu.SemaphoreType.DMA,
        ],
        compiler_params=pltpu.CompilerParams(needs_layout_passes=False),
    )
    def numer_kernel(er_hbm, ec_hbm, w_hbm, tbl_hbm, out_hbm,
                     cv, rbuf, wbuf, gbuf, acc, sem):
        cid = lax.axis_index("c")
        sid = lax.axis_index("s")
        wid = sid * NC + cid

        pltpu.sync_copy(ec_hbm.at[wid], cv)

        @pl.loop(0, K)
        def _zero_g(e):
            for j in range(D // L):
                gbuf[e, pl.ds(j * L, L)] = jnp.zeros((L,), jnp.float32)

        for z in range(rows_pt // K):
            pltpu.sync_copy(gbuf, acc.at[pl.ds(sid * rows_pt + z * K, K)])
        plsc.subcore_barrier()

        @pl.loop(0, nchunks)
        def _chunk(c):
            pltpu.sync_copy(er_hbm.at[wid, c], rbuf)
            pltpu.sync_copy(w_hbm.at[wid, c], wbuf)
            pltpu.async_copy(tbl_hbm.at[cv.at[c]], gbuf, sem).wait()

            @pl.loop(0, K)
            def _scale(e):
                ws = plsc.load_gather(wbuf, [jnp.full((L,), e, jnp.int32)])
                for j in range(D // L):
                    gbuf[e, pl.ds(j * L, L)] = gbuf[e, pl.ds(j * L, L)] * ws

            pltpu.sync_copy(gbuf, acc.at[rbuf], add=True)

        plsc.subcore_barrier()
        pltpu.sync_copy(acc.at[pl.ds(sid * rows_pt, rows_pt)],
                        out_hbm.at[cid, pl.ds(sid * rows_pt, rows_pt)])

    return numer_kernel


def _sparse(rr, cc, tbl, s1, s2, tbl_rows, nchunks):
    w, sparts = _make_weights(tbl_rows, nchunks)(rr, cc, s1, s2)
    parts = _make_numer(tbl_rows, nchunks)(rr, cc, w, tbl)
    return parts, sparts


def kernel(ent_em, rel_em, adj1_index, adj2_rows, adj2_cols, params):
    p = params
    f32 = jnp.float32

    x = jnp.pad(ent_em, ((0, NP - N), (0, 0)))
    xr = jnp.pad(rel_em, ((0, RP - R), (0, 0)))

    row2 = lambda a: a.reshape(1, D)

    value, s1, s2 = _dense1(
        x, p["la_w_k"], row2(p["la_w_b"]),
        p["la_w1_k"], row2(p["la_w1_b"]), p["la_w2_k"], row2(p["la_w2_b"]),
        row2(p["la_ln1_g"]), row2(p["la_ln1_b"]))

    r1 = adj1_index[0].reshape(NW, CH1, K)
    c1 = adj1_index[1].reshape(NW, CH1, K)
    parts1, sparts1 = _sparse(r1, c1, value, s1, s2, NP, CH1)

    e_attn, t1 = _dense2(
        parts1, sparts1, p["er_w1_k"], row2(p["er_w1_b"]),
        row2(p["la_ln2_g"]), row2(p["la_ln2_b"]))

    dtab, t2 = _dense_rel(
        xr, p["er_w_k"], row2(p["er_w_b"]), p["er_w2_k"], row2(p["er_w2_b"]),
        row2(p["er_ln1_g"]), row2(p["er_ln1_b"]))

    pad_r = jnp.full((E2P - E2,), N + 1, jnp.int32)
    pad_c = jnp.full((E2P - E2,), R, jnp.int32)
    r2 = jnp.concatenate([adj2_rows, pad_r]).reshape(NW, CH2, K)
    c2 = jnp.concatenate([adj2_cols, pad_c]).reshape(NW, CH2, K)
    parts2, sparts2 = _sparse(r2, c2, dtab, t1, t2, RP, CH2)

    res = _dense3(
        parts2, sparts2, e_attn, row2(p["er_ln2_g"]), row2(p["er_ln2_b"]),
        p["d1_k"], row2(p["d1_b"]))
    return res[:N]


# speedup vs baseline: 17.7448x; 1.0186x over previous
import functools

import jax
import jax.numpy as jnp
from jax import lax
from jax.experimental import pallas as pl
from jax.experimental.pallas import tpu as pltpu
from jax.experimental.pallas import tpu_sc as plsc

N = 10000
R = 1000
D = 128
E1 = 320000
E2 = 160000

NC, NS, L = 2, 16, 16
NW = NC * NS
K = 80
NP = 10240
RP = 1008
CH1 = E1 // (NW * K)
E2P = 161280
CH2 = E2P // (NW * K)
RB = 1024

_EPS = 1e-6


def _ln(x, g, b):
    m = jnp.mean(x, axis=1, keepdims=True)
    v = jnp.mean((x - m) ** 2, axis=1, keepdims=True)
    return (x - m) / jnp.sqrt(v + _EPS) * g + b


def _dense1_body(x_ref, wk_ref, wb_ref, w1_ref, b1_ref, w2_ref, b2_ref,
                 g_ref, b_ref, val_ref, s1_ref, s2_ref):
    x = x_ref[...]
    q = _ln(x, g_ref[...], b_ref[...])
    val_ref[...] = jnp.dot(q, wk_ref[...],
                           preferred_element_type=jnp.float32) + wb_ref[...]
    q1 = jnp.dot(q, w1_ref[...], preferred_element_type=jnp.float32) + b1_ref[...]
    q2 = jnp.dot(q, w2_ref[...], preferred_element_type=jnp.float32) + b2_ref[...]
    s1_ref[...] = jnp.tanh(jnp.sum(q1, axis=1))
    s2_ref[...] = jnp.tanh(jnp.sum(q2, axis=1))


def _dense1(x, wk, wb, w1, b1, w2, b2, g, b):
    return pl.pallas_call(
        _dense1_body,
        grid=(NP // RB,),
        in_specs=[
            pl.BlockSpec((RB, D), lambda i: (i, 0)),
            pl.BlockSpec((D, D), lambda i: (0, 0)),
            pl.BlockSpec((1, D), lambda i: (0, 0)),
            pl.BlockSpec((D, D), lambda i: (0, 0)),
            pl.BlockSpec((1, D), lambda i: (0, 0)),
            pl.BlockSpec((D, D), lambda i: (0, 0)),
            pl.BlockSpec((1, D), lambda i: (0, 0)),
            pl.BlockSpec((1, D), lambda i: (0, 0)),
            pl.BlockSpec((1, D), lambda i: (0, 0)),
        ],
        out_specs=[
            pl.BlockSpec((RB, D), lambda i: (i, 0)),
            pl.BlockSpec((RB,), lambda i: (i,)),
            pl.BlockSpec((RB,), lambda i: (i,)),
        ],
        out_shape=[
            jax.ShapeDtypeStruct((NP, D), jnp.float32),
            jax.ShapeDtypeStruct((NP,), jnp.float32),
            jax.ShapeDtypeStruct((NP,), jnp.float32),
        ],
    )(x, wk, wb, w1, b1, w2, b2, g, b)


def _combine(parts, s_parts):
    num = parts[0] + parts[1]
    den = jnp.sum(s_parts, axis=0)[:, None]
    pos = den > 0.0
    return jnp.where(pos, num / jnp.where(pos, den, 1.0), 0.0)


def _dense2_body(p_ref, ps_ref, w1_ref, b1_ref, g_ref, b_ref, e_ref, s1_ref):
    out1 = _combine(p_ref[...], ps_ref[...])
    e = _ln(out1, g_ref[...], b_ref[...])
    e_ref[...] = e
    qw = jnp.dot(e, w1_ref[...], preferred_element_type=jnp.float32) + b1_ref[...]
    s1_ref[...] = jnp.tanh(jnp.sum(qw * e, axis=1))


def _dense2(parts, s_parts, w1, b1, g, b):
    return pl.pallas_call(
        _dense2_body,
        grid=(NP // RB,),
        in_specs=[
            pl.BlockSpec((2, RB, D), lambda i: (0, i, 0)),
            pl.BlockSpec((NW, RB), lambda i: (0, i)),
            pl.BlockSpec((D, D), lambda i: (0, 0)),
            pl.BlockSpec((1, D), lambda i: (0, 0)),
            pl.BlockSpec((1, D), lambda i: (0, 0)),
            pl.BlockSpec((1, D), lambda i: (0, 0)),
        ],
        out_specs=[
            pl.BlockSpec((RB, D), lambda i: (i, 0)),
            pl.BlockSpec((RB,), lambda i: (i,)),
        ],
        out_shape=[
            jax.ShapeDtypeStruct((NP, D), jnp.float32),
            jax.ShapeDtypeStruct((NP,), jnp.float32),
        ],
    )(parts, s_parts, w1, b1, g, b)


def _dense_rel_body(x_ref, wk_ref, wb_ref, w2_ref, b2_ref, g_ref, b_ref,
                    d_ref, s2_ref):
    x = x_ref[...]
    inp = _ln(x, g_ref[...], b_ref[...])
    d = jnp.dot(inp, wk_ref[...], preferred_element_type=jnp.float32) + wb_ref[...]
    rowid = lax.broadcasted_iota(jnp.int32, (RP, 1), 0)
    d_ref[...] = jnp.where(rowid < R, d, 0.0)
    qw = jnp.dot(inp, w2_ref[...], preferred_element_type=jnp.float32) + b2_ref[...]
    s2_ref[...] = jnp.tanh(jnp.sum(qw * inp, axis=1))


def _dense_rel(x, wk, wb, w2, b2, g, b):
    return pl.pallas_call(
        _dense_rel_body,
        grid=(1,),
        in_specs=[
            pl.BlockSpec((RP, D), lambda i: (0, 0)),
            pl.BlockSpec((D, D), lambda i: (0, 0)),
            pl.BlockSpec((1, D), lambda i: (0, 0)),
            pl.BlockSpec((D, D), lambda i: (0, 0)),
            pl.BlockSpec((1, D), lambda i: (0, 0)),
            pl.BlockSpec((1, D), lambda i: (0, 0)),
            pl.BlockSpec((1, D), lambda i: (0, 0)),
        ],
        out_specs=[
            pl.BlockSpec((RP, D), lambda i: (0, 0)),
            pl.BlockSpec((RP,), lambda i: (0,)),
        ],
        out_shape=[
            jax.ShapeDtypeStruct((RP, D), jnp.float32),
            jax.ShapeDtypeStruct((RP,), jnp.float32),
        ],
    )(x, wk, wb, w2, b2, g, b)


def _dense3_body(p_ref, ps_ref, e_ref, g_ref, b_ref, dk_ref, db_ref, res_ref):
    out2 = _combine(p_ref[...], ps_ref[...])
    er = _ln(out2, g_ref[...], b_ref[...])
    res_ref[...] = e_ref[...] + 0.6 * (
        jnp.dot(er, dk_ref[...], preferred_element_type=jnp.float32) + db_ref[...])


def _dense3(parts, s_parts, e_attn, g, b, dk, db):
    return pl.pallas_call(
        _dense3_body,
        grid=(NP // RB,),
        in_specs=[
            pl.BlockSpec((2, RB, D), lambda i: (0, i, 0)),
            pl.BlockSpec((NW, RB), lambda i: (0, i)),
            pl.BlockSpec((RB, D), lambda i: (i, 0)),
            pl.BlockSpec((1, D), lambda i: (0, 0)),
            pl.BlockSpec((1, D), lambda i: (0, 0)),
            pl.BlockSpec((D, D), lambda i: (0, 0)),
            pl.BlockSpec((1, D), lambda i: (0, 0)),
        ],
        out_specs=pl.BlockSpec((RB, D), lambda i: (i, 0)),
        out_shape=jax.ShapeDtypeStruct((NP, D), jnp.float32),
    )(parts, s_parts, e_attn, g, b, dk, db)


def _mesh():
    return plsc.VectorSubcoreMesh(core_axis_name="c", subcore_axis_name="s",
                                  num_cores=NC, num_subcores=NS)


@functools.lru_cache(maxsize=None)
def _make_weights(tbl_rows, nchunks):
    @functools.partial(
        pl.kernel,
        out_type=[
            jax.ShapeDtypeStruct((NW, nchunks, K), jnp.float32),
            jax.ShapeDtypeStruct((NW, NP), jnp.float32),
        ],
        mesh=_mesh(),
        scratch_types=[
            pltpu.VMEM((nchunks, K), jnp.int32),
            pltpu.VMEM((nchunks, K), jnp.int32),
            pltpu.VMEM((NP,), jnp.float32),
            pltpu.VMEM((tbl_rows,), jnp.float32),
            pltpu.VMEM((nchunks, K), jnp.float32),
            pltpu.VMEM((NP,), jnp.float32),
        ],
        compiler_params=pltpu.CompilerParams(needs_layout_passes=False),
    )
    def weights_kernel(er_hbm, ec_hbm, s1_hbm, s2_hbm, w_hbm, outs_hbm,
                       rv, cv, s1v, s2v, wv, sv):
        cid = lax.axis_index("c")
        sid = lax.axis_index("s")
        wid = sid * NC + cid

        pltpu.sync_copy(er_hbm.at[wid], rv)
        pltpu.sync_copy(ec_hbm.at[wid], cv)
        pltpu.sync_copy(s1_hbm, s1v)
        pltpu.sync_copy(s2_hbm, s2v)

        @pl.loop(0, NP // L)
        def _zero_s(i):
            sv[pl.ds(i * L, L)] = jnp.zeros((L,), jnp.float32)

        lane = lax.iota(jnp.int32, L)
        dummy = jnp.int32(NP - 8)

        @pl.loop(0, nchunks)
        def _chunk(c):
            for i in range(K // L):
                r16 = rv[c, pl.ds(i * L, L)]
                c16 = cv[c, pl.ds(i * L, L)]
                ev = plsc.load_gather(s1v, [r16]) + plsc.load_gather(s2v, [c16])
                ev = jnp.maximum(ev, 0.2 * ev)
                w16 = jnp.exp(ev)
                wv[c, pl.ds(i * L, L)] = w16
                rs, wsr = plsc.sort_key_val(r16, w16)
                prev = rs.at[jnp.maximum(lane - 1, 0)].get(
                    mode="promise_in_bounds")
                nxt = rs.at[jnp.minimum(lane + 1, L - 1)].get(
                    mode="promise_in_bounds")
                first = (rs != prev) | (lane == 0)
                last = (rs != nxt) | (lane == L - 1)
                prefix = plsc.cumsum(wsr)
                start = plsc.cummax(jnp.where(first, lane, 0))
                pb = prefix.at[jnp.maximum(start - 1, 0)].get(
                    mode="promise_in_bounds")
                tot = prefix - jnp.where(start > 0, pb, 0.0)
                sidx = jnp.where(last, rs, dummy)
                plsc.addupdate_scatter(sv, [sidx], tot)

        pltpu.sync_copy(wv, w_hbm.at[wid])
        pltpu.sync_copy(sv, outs_hbm.at[wid])

    return weights_kernel


@functools.lru_cache(maxsize=None)
def _make_numer(tbl_rows, nchunks):
    rows_pt = NP // NS

    @functools.partial(
        pl.kernel,
        out_type=jax.ShapeDtypeStruct((NC, NP, D), jnp.float32),
        mesh=_mesh(),
        scratch_types=[
            pltpu.VMEM((nchunks, K), jnp.int32),
            pltpu.VMEM((K,), jnp.int32),
            pltpu.VMEM((K,), jnp.float32),
            pltpu.VMEM((K, D), jnp.float32),
            pltpu.VMEM_SHARED((NP, D), jnp.float32),
            pltpu.SemaphoreType.DMA,
        ],
        compiler_params=pltpu.CompilerParams(needs_layout_passes=False),
    )
    def numer_kernel(er_hbm, ec_hbm, w_hbm, tbl_hbm, out_hbm,
                     cv, rbuf, wbuf, gbuf, acc, sem):
        cid = lax.axis_index("c")
        sid = lax.axis_index("s")
        wid = sid * NC + cid

        pltpu.sync_copy(ec_hbm.at[wid], cv)

        @pl.loop(0, K)
        def _zero_g(e):
            for j in range(D // L):
                gbuf[e, pl.ds(j * L, L)] = jnp.zeros((L,), jnp.float32)

        for z in range(rows_pt // K):
            pltpu.sync_copy(gbuf, acc.at[pl.ds(sid * rows_pt + z * K, K)])
        plsc.subcore_barrier()

        @pl.loop(0, nchunks)
        def _chunk(c):
            pltpu.sync_copy(er_hbm.at[wid, c], rbuf)
            pltpu.sync_copy(w_hbm.at[wid, c], wbuf)
            pltpu.async_copy(tbl_hbm.at[cv.at[c]], gbuf, sem).wait()

            @pl.loop(0, K, unroll=8)
            def _scale(e):
                ws = plsc.load_gather(wbuf, [jnp.full((L,), e, jnp.int32)])
                for j in range(D // L):
                    gbuf[e, pl.ds(j * L, L)] = gbuf[e, pl.ds(j * L, L)] * ws

            pltpu.sync_copy(gbuf, acc.at[rbuf], add=True)

        plsc.subcore_barrier()
        pltpu.sync_copy(acc.at[pl.ds(sid * rows_pt, rows_pt)],
                        out_hbm.at[cid, pl.ds(sid * rows_pt, rows_pt)])

    return numer_kernel


def _sparse(rr, cc, tbl, s1, s2, tbl_rows, nchunks):
    w, sparts = _make_weights(tbl_rows, nchunks)(rr, cc, s1, s2)
    parts = _make_numer(tbl_rows, nchunks)(rr, cc, w, tbl)
    return parts, sparts


def kernel(ent_em, rel_em, adj1_index, adj2_rows, adj2_cols, params):
    p = params
    f32 = jnp.float32

    x = jnp.pad(ent_em, ((0, NP - N), (0, 0)))
    xr = jnp.pad(rel_em, ((0, RP - R), (0, 0)))

    row2 = lambda a: a.reshape(1, D)

    value, s1, s2 = _dense1(
        x, p["la_w_k"], row2(p["la_w_b"]),
        p["la_w1_k"], row2(p["la_w1_b"]), p["la_w2_k"], row2(p["la_w2_b"]),
        row2(p["la_ln1_g"]), row2(p["la_ln1_b"]))

    r1 = adj1_index[0].reshape(NW, CH1, K)
    c1 = adj1_index[1].reshape(NW, CH1, K)
    parts1, sparts1 = _sparse(r1, c1, value, s1, s2, NP, CH1)

    e_attn, t1 = _dense2(
        parts1, sparts1, p["er_w1_k"], row2(p["er_w1_b"]),
        row2(p["la_ln2_g"]), row2(p["la_ln2_b"]))

    dtab, t2 = _dense_rel(
        xr, p["er_w_k"], row2(p["er_w_b"]), p["er_w2_k"], row2(p["er_w2_b"]),
        row2(p["er_ln1_g"]), row2(p["er_ln1_b"]))

    pad_r = jnp.full((E2P - E2,), N + 1, jnp.int32)
    pad_c = jnp.full((E2P - E2,), R, jnp.int32)
    r2 = jnp.concatenate([adj2_rows, pad_r]).reshape(NW, CH2, K)
    c2 = jnp.concatenate([adj2_cols, pad_c]).reshape(NW, CH2, K)
    parts2, sparts2 = _sparse(r2, c2, dtab, t1, t2, RP, CH2)

    res = _dense3(
        parts2, sparts2, e_attn, row2(p["er_ln2_g"]), row2(p["er_ln2_b"]),
        p["d1_k"], row2(p["d1_b"]))
    return res[:N]


# speedup vs baseline: 30.9786x; 1.7458x over previous
import functools

import jax
import jax.numpy as jnp
from jax import lax
from jax.experimental import pallas as pl
from jax.experimental.pallas import tpu as pltpu
from jax.experimental.pallas import tpu_sc as plsc

N = 10000
R = 1000
D = 128
E1 = 320000
E2 = 160000

NC, NS, L = 2, 16, 16
NW = NC * NS
K = 80
NP = 10240
RP = 1008
CH1 = E1 // (NW * K)
E2P = 161280
CH2 = E2P // (NW * K)
RB = 1024

_EPS = 1e-6


def _ln(x, g, b):
    m = jnp.mean(x, axis=1, keepdims=True)
    v = jnp.mean((x - m) ** 2, axis=1, keepdims=True)
    return (x - m) / jnp.sqrt(v + _EPS) * g + b


def _dense1_body(x_ref, wk_ref, wb_ref, w1_ref, b1_ref, w2_ref, b2_ref,
                 g_ref, b_ref, val_ref, s1_ref, s2_ref):
    x = x_ref[...]
    q = _ln(x, g_ref[...], b_ref[...])
    val_ref[...] = jnp.dot(q, wk_ref[...],
                           preferred_element_type=jnp.float32) + wb_ref[...]
    q1 = jnp.dot(q, w1_ref[...], preferred_element_type=jnp.float32) + b1_ref[...]
    q2 = jnp.dot(q, w2_ref[...], preferred_element_type=jnp.float32) + b2_ref[...]
    s1_ref[...] = jnp.tanh(jnp.sum(q1, axis=1))
    s2_ref[...] = jnp.tanh(jnp.sum(q2, axis=1))


def _dense1(x, wk, wb, w1, b1, w2, b2, g, b):
    return pl.pallas_call(
        _dense1_body,
        grid=(NP // RB,),
        in_specs=[
            pl.BlockSpec((RB, D), lambda i: (i, 0)),
            pl.BlockSpec((D, D), lambda i: (0, 0)),
            pl.BlockSpec((1, D), lambda i: (0, 0)),
            pl.BlockSpec((D, D), lambda i: (0, 0)),
            pl.BlockSpec((1, D), lambda i: (0, 0)),
            pl.BlockSpec((D, D), lambda i: (0, 0)),
            pl.BlockSpec((1, D), lambda i: (0, 0)),
            pl.BlockSpec((1, D), lambda i: (0, 0)),
            pl.BlockSpec((1, D), lambda i: (0, 0)),
        ],
        out_specs=[
            pl.BlockSpec((RB, D), lambda i: (i, 0)),
            pl.BlockSpec((RB,), lambda i: (i,)),
            pl.BlockSpec((RB,), lambda i: (i,)),
        ],
        out_shape=[
            jax.ShapeDtypeStruct((NP, D), jnp.float32),
            jax.ShapeDtypeStruct((NP,), jnp.float32),
            jax.ShapeDtypeStruct((NP,), jnp.float32),
        ],
    )(x, wk, wb, w1, b1, w2, b2, g, b)


def _combine(parts, s_parts):
    num = parts[0] + parts[1]
    den = jnp.sum(s_parts, axis=0)[:, None]
    pos = den > 0.0
    return jnp.where(pos, num / jnp.where(pos, den, 1.0), 0.0)


def _dense2_body(p_ref, ps_ref, w1_ref, b1_ref, g_ref, b_ref, e_ref, s1_ref):
    out1 = _combine(p_ref[...], ps_ref[...])
    e = _ln(out1, g_ref[...], b_ref[...])
    e_ref[...] = e
    qw = jnp.dot(e, w1_ref[...], preferred_element_type=jnp.float32) + b1_ref[...]
    s1_ref[...] = jnp.tanh(jnp.sum(qw * e, axis=1))


def _dense2(parts, s_parts, w1, b1, g, b):
    return pl.pallas_call(
        _dense2_body,
        grid=(NP // RB,),
        in_specs=[
            pl.BlockSpec((2, RB, D), lambda i: (0, i, 0)),
            pl.BlockSpec((NW, RB), lambda i: (0, i)),
            pl.BlockSpec((D, D), lambda i: (0, 0)),
            pl.BlockSpec((1, D), lambda i: (0, 0)),
            pl.BlockSpec((1, D), lambda i: (0, 0)),
            pl.BlockSpec((1, D), lambda i: (0, 0)),
        ],
        out_specs=[
            pl.BlockSpec((RB, D), lambda i: (i, 0)),
            pl.BlockSpec((RB,), lambda i: (i,)),
        ],
        out_shape=[
            jax.ShapeDtypeStruct((NP, D), jnp.float32),
            jax.ShapeDtypeStruct((NP,), jnp.float32),
        ],
    )(parts, s_parts, w1, b1, g, b)


def _dense_rel_body(x_ref, wk_ref, wb_ref, w2_ref, b2_ref, g_ref, b_ref,
                    d_ref, s2_ref):
    x = x_ref[...]
    inp = _ln(x, g_ref[...], b_ref[...])
    d = jnp.dot(inp, wk_ref[...], preferred_element_type=jnp.float32) + wb_ref[...]
    rowid = lax.broadcasted_iota(jnp.int32, (RP, 1), 0)
    d_ref[...] = jnp.where(rowid < R, d, 0.0)
    qw = jnp.dot(inp, w2_ref[...], preferred_element_type=jnp.float32) + b2_ref[...]
    s2_ref[...] = jnp.tanh(jnp.sum(qw * inp, axis=1))


def _dense_rel(x, wk, wb, w2, b2, g, b):
    return pl.pallas_call(
        _dense_rel_body,
        grid=(1,),
        in_specs=[
            pl.BlockSpec((RP, D), lambda i: (0, 0)),
            pl.BlockSpec((D, D), lambda i: (0, 0)),
            pl.BlockSpec((1, D), lambda i: (0, 0)),
            pl.BlockSpec((D, D), lambda i: (0, 0)),
            pl.BlockSpec((1, D), lambda i: (0, 0)),
            pl.BlockSpec((1, D), lambda i: (0, 0)),
            pl.BlockSpec((1, D), lambda i: (0, 0)),
        ],
        out_specs=[
            pl.BlockSpec((RP, D), lambda i: (0, 0)),
            pl.BlockSpec((RP,), lambda i: (0,)),
        ],
        out_shape=[
            jax.ShapeDtypeStruct((RP, D), jnp.float32),
            jax.ShapeDtypeStruct((RP,), jnp.float32),
        ],
    )(x, wk, wb, w2, b2, g, b)


def _dense3_body(p_ref, ps_ref, e_ref, g_ref, b_ref, dk_ref, db_ref, res_ref):
    out2 = _combine(p_ref[...], ps_ref[...])
    er = _ln(out2, g_ref[...], b_ref[...])
    res_ref[...] = e_ref[...] + 0.6 * (
        jnp.dot(er, dk_ref[...], preferred_element_type=jnp.float32) + db_ref[...])


def _dense3(parts, s_parts, e_attn, g, b, dk, db):
    return pl.pallas_call(
        _dense3_body,
        grid=(NP // RB,),
        in_specs=[
            pl.BlockSpec((2, RB, D), lambda i: (0, i, 0)),
            pl.BlockSpec((NW, RB), lambda i: (0, i)),
            pl.BlockSpec((RB, D), lambda i: (i, 0)),
            pl.BlockSpec((1, D), lambda i: (0, 0)),
            pl.BlockSpec((1, D), lambda i: (0, 0)),
            pl.BlockSpec((D, D), lambda i: (0, 0)),
            pl.BlockSpec((1, D), lambda i: (0, 0)),
        ],
        out_specs=pl.BlockSpec((RB, D), lambda i: (i, 0)),
        out_shape=jax.ShapeDtypeStruct((NP, D), jnp.float32),
    )(parts, s_parts, e_attn, g, b, dk, db)


def _mesh():
    return plsc.VectorSubcoreMesh(core_axis_name="c", subcore_axis_name="s",
                                  num_cores=NC, num_subcores=NS)


@functools.lru_cache(maxsize=None)
def _make_weights(tbl_rows, nchunks):
    @functools.partial(
        pl.kernel,
        out_type=[
            jax.ShapeDtypeStruct((NW, nchunks, K), jnp.float32),
            jax.ShapeDtypeStruct((NW, NP), jnp.float32),
        ],
        mesh=_mesh(),
        scratch_types=[
            pltpu.VMEM((nchunks, K), jnp.int32),
            pltpu.VMEM((nchunks, K), jnp.int32),
            pltpu.VMEM((NP,), jnp.float32),
            pltpu.VMEM((tbl_rows,), jnp.float32),
            pltpu.VMEM((nchunks, K), jnp.float32),
            pltpu.VMEM((NP,), jnp.float32),
        ],
        compiler_params=pltpu.CompilerParams(needs_layout_passes=False),
    )
    def weights_kernel(er_hbm, ec_hbm, s1_hbm, s2_hbm, w_hbm, outs_hbm,
                       rv, cv, s1v, s2v, wv, sv):
        cid = lax.axis_index("c")
        sid = lax.axis_index("s")
        wid = sid * NC + cid

        pltpu.sync_copy(er_hbm.at[wid], rv)
        pltpu.sync_copy(ec_hbm.at[wid], cv)
        pltpu.sync_copy(s1_hbm, s1v)
        pltpu.sync_copy(s2_hbm, s2v)

        @pl.loop(0, NP // L)
        def _zero_s(i):
            sv[pl.ds(i * L, L)] = jnp.zeros((L,), jnp.float32)

        lane = lax.iota(jnp.int32, L)
        dummy = jnp.int32(NP - 8)

        @pl.loop(0, nchunks)
        def _chunk(c):
            for i in range(K // L):
                r16 = rv[c, pl.ds(i * L, L)]
                c16 = cv[c, pl.ds(i * L, L)]
                ev = plsc.load_gather(s1v, [r16]) + plsc.load_gather(s2v, [c16])
                ev = jnp.maximum(ev, 0.2 * ev)
                w16 = jnp.exp(ev)
                wv[c, pl.ds(i * L, L)] = w16
                rs, wsr = plsc.sort_key_val(r16, w16)
                prev = rs.at[jnp.maximum(lane - 1, 0)].get(
                    mode="promise_in_bounds")
                nxt = rs.at[jnp.minimum(lane + 1, L - 1)].get(
                    mode="promise_in_bounds")
                first = (rs != prev) | (lane == 0)
                last = (rs != nxt) | (lane == L - 1)
                prefix = plsc.cumsum(wsr)
                start = plsc.cummax(jnp.where(first, lane, 0))
                pb = prefix.at[jnp.maximum(start - 1, 0)].get(
                    mode="promise_in_bounds")
                tot = prefix - jnp.where(start > 0, pb, 0.0)
                sidx = jnp.where(last, rs, dummy)
                plsc.addupdate_scatter(sv, [sidx], tot)

        pltpu.sync_copy(wv, w_hbm.at[wid])
        pltpu.sync_copy(sv, outs_hbm.at[wid])

    return weights_kernel


@functools.lru_cache(maxsize=None)
def _make_numer(tbl_rows, nchunks):
    rows_pt = NP // NS

    @functools.partial(
        pl.kernel,
        out_type=jax.ShapeDtypeStruct((NC, NP, D), jnp.float32),
        mesh=_mesh(),
        scratch_types=[
            pltpu.VMEM((nchunks, K), jnp.int32),
            pltpu.VMEM((K,), jnp.int32),
            pltpu.VMEM((K,), jnp.int32),
            pltpu.VMEM((K,), jnp.float32),
            pltpu.VMEM((K,), jnp.float32),
            pltpu.VMEM((K, D), jnp.float32),
            pltpu.VMEM((K, D), jnp.float32),
            pltpu.VMEM_SHARED((NP, D), jnp.float32),
            pltpu.SemaphoreType.DMA,
            pltpu.SemaphoreType.DMA,
        ],
        compiler_params=pltpu.CompilerParams(needs_layout_passes=False),
    )
    def numer_kernel(er_hbm, ec_hbm, w_hbm, tbl_hbm, out_hbm,
                     cv, rbufa, rbufb, wbufa, wbufb, gbufa, gbufb,
                     acc, sema, semb):
        cid = lax.axis_index("c")
        sid = lax.axis_index("s")
        wid = sid * NC + cid

        pltpu.sync_copy(ec_hbm.at[wid], cv)

        @pl.loop(0, K)
        def _zero_g(e):
            for j in range(D // L):
                gbufa[e, pl.ds(j * L, L)] = jnp.zeros((L,), jnp.float32)

        for z in range(rows_pt // K):
            pltpu.sync_copy(gbufa, acc.at[pl.ds(sid * rows_pt + z * K, K)])
        plsc.subcore_barrier()

        def issue(c, rb, wb, gb, sem):
            base = (wid * nchunks + c) * K
            pltpu.async_copy(er_hbm.at[pl.ds(base, K)], rb, sem)
            pltpu.async_copy(w_hbm.at[pl.ds(base, K)], wb, sem)
            pltpu.async_copy(tbl_hbm.at[cv.at[c]], gb, sem)

        def wait(c, rb, wb, gb, sem):
            base = (wid * nchunks + c) * K
            pltpu.make_async_copy(er_hbm.at[pl.ds(base, K)], rb, sem).wait()
            pltpu.make_async_copy(w_hbm.at[pl.ds(base, K)], wb, sem).wait()
            pltpu.make_async_copy(tbl_hbm.at[cv.at[c]], gb, sem).wait()

        def process(rb, wb, gb):
            @pl.loop(0, K, unroll=8)
            def _scale(e):
                ws = plsc.load_gather(wb, [jnp.full((L,), e, jnp.int32)])
                for j in range(D // L):
                    gb[e, pl.ds(j * L, L)] = gb[e, pl.ds(j * L, L)] * ws

            pltpu.sync_copy(gb, acc.at[rb], add=True)

        assert nchunks % 2 == 1
        issue(0, rbufa, wbufa, gbufa, sema)

        @pl.loop(0, (nchunks - 1) // 2)
        def _pair(t):
            ca = 2 * t
            cb = 2 * t + 1
            issue(cb, rbufb, wbufb, gbufb, semb)
            wait(ca, rbufa, wbufa, gbufa, sema)
            process(rbufa, wbufa, gbufa)
            issue(ca + 2, rbufa, wbufa, gbufa, sema)
            wait(cb, rbufb, wbufb, gbufb, semb)
            process(rbufb, wbufb, gbufb)

        wait(nchunks - 1, rbufa, wbufa, gbufa, sema)
        process(rbufa, wbufa, gbufa)

        plsc.subcore_barrier()
        pltpu.sync_copy(acc.at[pl.ds(sid * rows_pt, rows_pt)],
                        out_hbm.at[cid, pl.ds(sid * rows_pt, rows_pt)])

    return numer_kernel


def _sparse(rr, cc, tbl, s1, s2, tbl_rows, nchunks):
    w, sparts = _make_weights(tbl_rows, nchunks)(rr, cc, s1, s2)
    parts = _make_numer(tbl_rows, nchunks)(rr.reshape(-1), cc,
                                           w.reshape(-1), tbl)
    return parts, sparts


def kernel(ent_em, rel_em, adj1_index, adj2_rows, adj2_cols, params):
    p = params
    f32 = jnp.float32

    x = jnp.pad(ent_em, ((0, NP - N), (0, 0)))
    xr = jnp.pad(rel_em, ((0, RP - R), (0, 0)))

    row2 = lambda a: a.reshape(1, D)

    value, s1, s2 = _dense1(
        x, p["la_w_k"], row2(p["la_w_b"]),
        p["la_w1_k"], row2(p["la_w1_b"]), p["la_w2_k"], row2(p["la_w2_b"]),
        row2(p["la_ln1_g"]), row2(p["la_ln1_b"]))

    r1 = adj1_index[0].reshape(NW, CH1, K)
    c1 = adj1_index[1].reshape(NW, CH1, K)
    parts1, sparts1 = _sparse(r1, c1, value, s1, s2, NP, CH1)

    e_attn, t1 = _dense2(
        parts1, sparts1, p["er_w1_k"], row2(p["er_w1_b"]),
        row2(p["la_ln2_g"]), row2(p["la_ln2_b"]))

    dtab, t2 = _dense_rel(
        xr, p["er_w_k"], row2(p["er_w_b"]), p["er_w2_k"], row2(p["er_w2_b"]),
        row2(p["er_ln1_g"]), row2(p["er_ln1_b"]))

    pad_r = jnp.full((E2P - E2,), N + 1, jnp.int32)
    pad_c = jnp.full((E2P - E2,), R, jnp.int32)
    r2 = jnp.concatenate([adj2_rows, pad_r]).reshape(NW, CH2, K)
    c2 = jnp.concatenate([adj2_cols, pad_c]).reshape(NW, CH2, K)
    parts2, sparts2 = _sparse(r2, c2, dtab, t1, t2, RP, CH2)

    res = _dense3(
        parts2, sparts2, e_attn, row2(p["er_ln2_g"]), row2(p["er_ln2_b"]),
        p["d1_k"], row2(p["d1_b"]))
    return res[:N]


# speedup vs baseline: 30.9846x; 1.0002x over previous
import functools

import jax
import jax.numpy as jnp
from jax import lax
from jax.experimental import pallas as pl
from jax.experimental.pallas import tpu as pltpu
from jax.experimental.pallas import tpu_sc as plsc

N = 10000
R = 1000
D = 128
E1 = 320000
E2 = 160000

NC, NS, L = 2, 16, 16
NW = NC * NS
K = 80
NP = 10240
RP = 1024
CH1 = E1 // (NW * K)
E2P = 161280
CH2 = E2P // (NW * K)
RB = 1024

_EPS = 1e-6


def _ln(x, g, b):
    m = jnp.mean(x, axis=1, keepdims=True)
    v = jnp.mean((x - m) ** 2, axis=1, keepdims=True)
    return (x - m) / jnp.sqrt(v + _EPS) * g + b


def _dense1_body(x_ref, wk_ref, wb_ref, w1_ref, b1_ref, w2_ref, b2_ref,
                 g_ref, b_ref, val_ref, s1_ref, s2_ref):
    x = x_ref[...]
    q = _ln(x, g_ref[...], b_ref[...])
    val_ref[...] = jnp.dot(q, wk_ref[...],
                           preferred_element_type=jnp.float32) + wb_ref[...]
    q1 = jnp.dot(q, w1_ref[...], preferred_element_type=jnp.float32) + b1_ref[...]
    q2 = jnp.dot(q, w2_ref[...], preferred_element_type=jnp.float32) + b2_ref[...]
    s1_ref[...] = jnp.tanh(jnp.sum(q1, axis=1))
    s2_ref[...] = jnp.tanh(jnp.sum(q2, axis=1))


def _dense1(x, wk, wb, w1, b1, w2, b2, g, b):
    return pl.pallas_call(
        _dense1_body,
        grid=(NP // RB,),
        in_specs=[
            pl.BlockSpec((RB, D), lambda i: (i, 0)),
            pl.BlockSpec((D, D), lambda i: (0, 0)),
            pl.BlockSpec((1, D), lambda i: (0, 0)),
            pl.BlockSpec((D, D), lambda i: (0, 0)),
            pl.BlockSpec((1, D), lambda i: (0, 0)),
            pl.BlockSpec((D, D), lambda i: (0, 0)),
            pl.BlockSpec((1, D), lambda i: (0, 0)),
            pl.BlockSpec((1, D), lambda i: (0, 0)),
            pl.BlockSpec((1, D), lambda i: (0, 0)),
        ],
        out_specs=[
            pl.BlockSpec((RB, D), lambda i: (i, 0)),
            pl.BlockSpec((RB,), lambda i: (i,)),
            pl.BlockSpec((RB,), lambda i: (i,)),
        ],
        out_shape=[
            jax.ShapeDtypeStruct((NP, D), jnp.float32),
            jax.ShapeDtypeStruct((NP,), jnp.float32),
            jax.ShapeDtypeStruct((NP,), jnp.float32),
        ],
    )(x, wk, wb, w1, b1, w2, b2, g, b)


def _combine(parts, s_parts):
    num = parts[0] + parts[1]
    den = jnp.sum(s_parts, axis=0)[:, None]
    pos = den > 0.0
    return jnp.where(pos, num / jnp.where(pos, den, 1.0), 0.0)


def _dense2_body(p_ref, ps_ref, w1_ref, b1_ref, g_ref, b_ref, e_ref, s1_ref):
    out1 = _combine(p_ref[...], ps_ref[...])
    e = _ln(out1, g_ref[...], b_ref[...])
    e_ref[...] = e
    qw = jnp.dot(e, w1_ref[...], preferred_element_type=jnp.float32) + b1_ref[...]
    s1_ref[...] = jnp.tanh(jnp.sum(qw * e, axis=1))


def _dense2(parts, s_parts, w1, b1, g, b):
    return pl.pallas_call(
        _dense2_body,
        grid=(NP // RB,),
        in_specs=[
            pl.BlockSpec((2, RB, D), lambda i: (0, i, 0)),
            pl.BlockSpec((NW, RB), lambda i: (0, i)),
            pl.BlockSpec((D, D), lambda i: (0, 0)),
            pl.BlockSpec((1, D), lambda i: (0, 0)),
            pl.BlockSpec((1, D), lambda i: (0, 0)),
            pl.BlockSpec((1, D), lambda i: (0, 0)),
        ],
        out_specs=[
            pl.BlockSpec((RB, D), lambda i: (i, 0)),
            pl.BlockSpec((RB,), lambda i: (i,)),
        ],
        out_shape=[
            jax.ShapeDtypeStruct((NP, D), jnp.float32),
            jax.ShapeDtypeStruct((NP,), jnp.float32),
        ],
    )(parts, s_parts, w1, b1, g, b)


def _dense_rel_body(x_ref, wk_ref, wb_ref, w2_ref, b2_ref, g_ref, b_ref,
                    d_ref, s2_ref):
    x = x_ref[...]
    inp = _ln(x, g_ref[...], b_ref[...])
    d = jnp.dot(inp, wk_ref[...], preferred_element_type=jnp.float32) + wb_ref[...]
    rowid = lax.broadcasted_iota(jnp.int32, (RP, 1), 0)
    d_ref[...] = jnp.where(rowid < R, d, 0.0)
    qw = jnp.dot(inp, w2_ref[...], preferred_element_type=jnp.float32) + b2_ref[...]
    s2_ref[...] = jnp.tanh(jnp.sum(qw * inp, axis=1))


def _dense_rel(x, wk, wb, w2, b2, g, b):
    return pl.pallas_call(
        _dense_rel_body,
        grid=(1,),
        in_specs=[
            pl.BlockSpec((RP, D), lambda i: (0, 0)),
            pl.BlockSpec((D, D), lambda i: (0, 0)),
            pl.BlockSpec((1, D), lambda i: (0, 0)),
            pl.BlockSpec((D, D), lambda i: (0, 0)),
            pl.BlockSpec((1, D), lambda i: (0, 0)),
            pl.BlockSpec((1, D), lambda i: (0, 0)),
            pl.BlockSpec((1, D), lambda i: (0, 0)),
        ],
        out_specs=[
            pl.BlockSpec((RP, D), lambda i: (0, 0)),
            pl.BlockSpec((RP,), lambda i: (0,)),
        ],
        out_shape=[
            jax.ShapeDtypeStruct((RP, D), jnp.float32),
            jax.ShapeDtypeStruct((RP,), jnp.float32),
        ],
    )(x, wk, wb, w2, b2, g, b)


def _dense3_body(p_ref, ps_ref, e_ref, g_ref, b_ref, dk_ref, db_ref, res_ref):
    out2 = _combine(p_ref[...], ps_ref[...])
    er = _ln(out2, g_ref[...], b_ref[...])
    res_ref[...] = e_ref[...] + 0.6 * (
        jnp.dot(er, dk_ref[...], preferred_element_type=jnp.float32) + db_ref[...])


def _dense3(parts, s_parts, e_attn, g, b, dk, db):
    return pl.pallas_call(
        _dense3_body,
        grid=(NP // RB,),
        in_specs=[
            pl.BlockSpec((2, RB, D), lambda i: (0, i, 0)),
            pl.BlockSpec((NW, RB), lambda i: (0, i)),
            pl.BlockSpec((RB, D), lambda i: (i, 0)),
            pl.BlockSpec((1, D), lambda i: (0, 0)),
            pl.BlockSpec((1, D), lambda i: (0, 0)),
            pl.BlockSpec((D, D), lambda i: (0, 0)),
            pl.BlockSpec((1, D), lambda i: (0, 0)),
        ],
        out_specs=pl.BlockSpec((RB, D), lambda i: (i, 0)),
        out_shape=jax.ShapeDtypeStruct((NP, D), jnp.float32),
    )(parts, s_parts, e_attn, g, b, dk, db)


def _mesh():
    return plsc.VectorSubcoreMesh(core_axis_name="c", subcore_axis_name="s",
                                  num_cores=NC, num_subcores=NS)


@functools.lru_cache(maxsize=None)
def _make_weights(tbl_rows, nchunks):
    @functools.partial(
        pl.kernel,
        out_type=[
            jax.ShapeDtypeStruct((NW, nchunks, K), jnp.float32),
            jax.ShapeDtypeStruct((NW, NP), jnp.float32),
        ],
        mesh=_mesh(),
        scratch_types=[
            pltpu.VMEM((nchunks, K), jnp.int32),
            pltpu.VMEM((nchunks, K), jnp.int32),
            pltpu.VMEM((NP,), jnp.float32),
            pltpu.VMEM((tbl_rows,), jnp.float32),
            pltpu.VMEM((nchunks, K), jnp.float32),
            pltpu.VMEM((NP,), jnp.float32),
        ],
        compiler_params=pltpu.CompilerParams(needs_layout_passes=False),
    )
    def weights_kernel(er_hbm, ec_hbm, s1_hbm, s2_hbm, w_hbm, outs_hbm,
                       rv, cv, s1v, s2v, wv, sv):
        cid = lax.axis_index("c")
        sid = lax.axis_index("s")
        wid = sid * NC + cid

        pltpu.sync_copy(er_hbm.at[wid], rv)
        pltpu.sync_copy(ec_hbm.at[wid], cv)
        pltpu.sync_copy(s1_hbm, s1v)
        pltpu.sync_copy(s2_hbm, s2v)

        @pl.loop(0, NP // L)
        def _zero_s(i):
            sv[pl.ds(i * L, L)] = jnp.zeros((L,), jnp.float32)

        lane = lax.iota(jnp.int32, L)
        dummy = jnp.int32(NP - 8)

        @pl.loop(0, nchunks)
        def _chunk(c):
            for i in range(K // L):
                r16 = rv[c, pl.ds(i * L, L)]
                c16 = cv[c, pl.ds(i * L, L)]
                ev = plsc.load_gather(s1v, [r16]) + plsc.load_gather(s2v, [c16])
                ev = jnp.maximum(ev, 0.2 * ev)
                w16 = jnp.exp(ev)
                wv[c, pl.ds(i * L, L)] = w16
                rs, wsr = plsc.sort_key_val(r16, w16)
                prev = rs.at[jnp.maximum(lane - 1, 0)].get(
                    mode="promise_in_bounds")
                nxt = rs.at[jnp.minimum(lane + 1, L - 1)].get(
                    mode="promise_in_bounds")
                first = (rs != prev) | (lane == 0)
                last = (rs != nxt) | (lane == L - 1)
                prefix = plsc.cumsum(wsr)
                start = plsc.cummax(jnp.where(first, lane, 0))
                pb = prefix.at[jnp.maximum(start - 1, 0)].get(
                    mode="promise_in_bounds")
                tot = prefix - jnp.where(start > 0, pb, 0.0)
                sidx = jnp.where(last, rs, dummy)
                plsc.addupdate_scatter(sv, [sidx], tot)

        pltpu.sync_copy(wv, w_hbm.at[wid])
        pltpu.sync_copy(sv, outs_hbm.at[wid])

    return weights_kernel


@functools.lru_cache(maxsize=None)
def _make_numer(tbl_rows, nchunks, tbl_in_spmem=False):
    rows_pt = NP // NS
    tbl_pt = tbl_rows // NS

    scratch = [
        pltpu.VMEM((nchunks, K), jnp.int32),
        pltpu.VMEM((K,), jnp.int32),
        pltpu.VMEM((K,), jnp.int32),
        pltpu.VMEM((K,), jnp.float32),
        pltpu.VMEM((K,), jnp.float32),
        pltpu.VMEM((K, D), jnp.float32),
        pltpu.VMEM((K, D), jnp.float32),
        pltpu.VMEM_SHARED((NP, D), jnp.float32),
        pltpu.SemaphoreType.DMA,
        pltpu.SemaphoreType.DMA,
    ]
    if tbl_in_spmem:
        scratch.append(pltpu.VMEM_SHARED((tbl_rows, D), jnp.float32))

    @functools.partial(
        pl.kernel,
        out_type=jax.ShapeDtypeStruct((NC, NP, D), jnp.float32),
        mesh=_mesh(),
        scratch_types=scratch,
        compiler_params=pltpu.CompilerParams(needs_layout_passes=False),
    )
    def numer_kernel(er_hbm, ec_hbm, w_hbm, tbl_hbm, out_hbm,
                     cv, rbufa, rbufb, wbufa, wbufb, gbufa, gbufb,
                     acc, sema, semb, *maybe_tbl):
        cid = lax.axis_index("c")
        sid = lax.axis_index("s")
        wid = sid * NC + cid

        pltpu.sync_copy(ec_hbm.at[wid], cv)

        if tbl_in_spmem:
            tbl = maybe_tbl[0]
            pltpu.sync_copy(tbl_hbm.at[pl.ds(sid * tbl_pt, tbl_pt)],
                            tbl.at[pl.ds(sid * tbl_pt, tbl_pt)])
        else:
            tbl = tbl_hbm

        @pl.loop(0, K)
        def _zero_g(e):
            for j in range(D // L):
                gbufa[e, pl.ds(j * L, L)] = jnp.zeros((L,), jnp.float32)

        for z in range(rows_pt // K):
            pltpu.sync_copy(gbufa, acc.at[pl.ds(sid * rows_pt + z * K, K)])
        plsc.subcore_barrier()

        def issue(c, rb, wb, gb, sem):
            base = (wid * nchunks + c) * K
            pltpu.async_copy(er_hbm.at[pl.ds(base, K)], rb, sem)
            pltpu.async_copy(w_hbm.at[pl.ds(base, K)], wb, sem)
            pltpu.async_copy(tbl.at[cv.at[c]], gb, sem)

        def wait(c, rb, wb, gb, sem):
            base = (wid * nchunks + c) * K
            pltpu.make_async_copy(er_hbm.at[pl.ds(base, K)], rb, sem).wait()
            pltpu.make_async_copy(w_hbm.at[pl.ds(base, K)], wb, sem).wait()
            pltpu.make_async_copy(tbl.at[cv.at[c]], gb, sem).wait()

        def process(rb, wb, gb):
            @pl.loop(0, K, unroll=8)
            def _scale(e):
                ws = plsc.load_gather(wb, [jnp.full((L,), e, jnp.int32)])
                for j in range(D // L):
                    gb[e, pl.ds(j * L, L)] = gb[e, pl.ds(j * L, L)] * ws

            pltpu.sync_copy(gb, acc.at[rb], add=True)

        assert nchunks % 2 == 1
        issue(0, rbufa, wbufa, gbufa, sema)

        @pl.loop(0, (nchunks - 1) // 2)
        def _pair(t):
            ca = 2 * t
            cb = 2 * t + 1
            issue(cb, rbufb, wbufb, gbufb, semb)
            wait(ca, rbufa, wbufa, gbufa, sema)
            process(rbufa, wbufa, gbufa)
            issue(ca + 2, rbufa, wbufa, gbufa, sema)
            wait(cb, rbufb, wbufb, gbufb, semb)
            process(rbufb, wbufb, gbufb)

        wait(nchunks - 1, rbufa, wbufa, gbufa, sema)
        process(rbufa, wbufa, gbufa)

        plsc.subcore_barrier()
        pltpu.sync_copy(acc.at[pl.ds(sid * rows_pt, rows_pt)],
                        out_hbm.at[cid, pl.ds(sid * rows_pt, rows_pt)])

    return numer_kernel


def _sparse(rr, cc, tbl, s1, s2, tbl_rows, nchunks, tbl_in_spmem=False):
    w, sparts = _make_weights(tbl_rows, nchunks)(rr, cc, s1, s2)
    parts = _make_numer(tbl_rows, nchunks, tbl_in_spmem)(rr.reshape(-1), cc,
                                                         w.reshape(-1), tbl)
    return parts, sparts


def kernel(ent_em, rel_em, adj1_index, adj2_rows, adj2_cols, params):
    p = params
    f32 = jnp.float32

    x = jnp.pad(ent_em, ((0, NP - N), (0, 0)))
    xr = jnp.pad(rel_em, ((0, RP - R), (0, 0)))

    row2 = lambda a: a.reshape(1, D)

    value, s1, s2 = _dense1(
        x, p["la_w_k"], row2(p["la_w_b"]),
        p["la_w1_k"], row2(p["la_w1_b"]), p["la_w2_k"], row2(p["la_w2_b"]),
        row2(p["la_ln1_g"]), row2(p["la_ln1_b"]))

    r1 = adj1_index[0].reshape(NW, CH1, K)
    c1 = adj1_index[1].reshape(NW, CH1, K)
    parts1, sparts1 = _sparse(r1, c1, value, s1, s2, NP, CH1)

    e_attn, t1 = _dense2(
        parts1, sparts1, p["er_w1_k"], row2(p["er_w1_b"]),
        row2(p["la_ln2_g"]), row2(p["la_ln2_b"]))

    dtab, t2 = _dense_rel(
        xr, p["er_w_k"], row2(p["er_w_b"]), p["er_w2_k"], row2(p["er_w2_b"]),
        row2(p["er_ln1_g"]), row2(p["er_ln1_b"]))

    pad_r = jnp.full((E2P - E2,), N + 1, jnp.int32)
    pad_c = jnp.full((E2P - E2,), R, jnp.int32)
    r2 = jnp.concatenate([adj2_rows, pad_r]).reshape(NW, CH2, K)
    c2 = jnp.concatenate([adj2_cols, pad_c]).reshape(NW, CH2, K)
    parts2, sparts2 = _sparse(r2, c2, dtab, t1, t2, RP, CH2)

    res = _dense3(
        parts2, sparts2, e_attn, row2(p["er_ln2_g"]), row2(p["er_ln2_b"]),
        p["d1_k"], row2(p["d1_b"]))
    return res[:N]


# speedup vs baseline: 30.9919x; 1.0002x over previous
import functools

import jax
import jax.numpy as jnp
from jax import lax
from jax.experimental import pallas as pl
from jax.experimental.pallas import tpu as pltpu
from jax.experimental.pallas import tpu_sc as plsc

N = 10000
R = 1000
D = 128
E1 = 320000
E2 = 160000

NC, NS, L = 2, 16, 16
NW = NC * NS
K = 80
NP = 10240
RP = 1024
CH1 = E1 // (NW * K)
E2P = 161280
CH2 = E2P // (NW * K)
RB = 1024

_EPS = 1e-6


def _ln(x, g, b):
    m = jnp.mean(x, axis=1, keepdims=True)
    v = jnp.mean((x - m) ** 2, axis=1, keepdims=True)
    return (x - m) / jnp.sqrt(v + _EPS) * g + b


def _dense1_body(x_ref, wk_ref, wb_ref, w1_ref, b1_ref, w2_ref, b2_ref,
                 g_ref, b_ref, val_ref, s1_ref, s2_ref):
    x = x_ref[...]
    q = _ln(x, g_ref[...], b_ref[...])
    val_ref[...] = jnp.dot(q, wk_ref[...],
                           preferred_element_type=jnp.float32) + wb_ref[...]
    q1 = jnp.dot(q, w1_ref[...], preferred_element_type=jnp.float32) + b1_ref[...]
    q2 = jnp.dot(q, w2_ref[...], preferred_element_type=jnp.float32) + b2_ref[...]
    s1_ref[...] = jnp.tanh(jnp.sum(q1, axis=1))
    s2_ref[...] = jnp.tanh(jnp.sum(q2, axis=1))


def _dense1(x, wk, wb, w1, b1, w2, b2, g, b):
    return pl.pallas_call(
        _dense1_body,
        grid=(NP // RB,),
        in_specs=[
            pl.BlockSpec((RB, D), lambda i: (i, 0)),
            pl.BlockSpec((D, D), lambda i: (0, 0)),
            pl.BlockSpec((1, D), lambda i: (0, 0)),
            pl.BlockSpec((D, D), lambda i: (0, 0)),
            pl.BlockSpec((1, D), lambda i: (0, 0)),
            pl.BlockSpec((D, D), lambda i: (0, 0)),
            pl.BlockSpec((1, D), lambda i: (0, 0)),
            pl.BlockSpec((1, D), lambda i: (0, 0)),
            pl.BlockSpec((1, D), lambda i: (0, 0)),
        ],
        out_specs=[
            pl.BlockSpec((RB, D), lambda i: (i, 0)),
            pl.BlockSpec((RB,), lambda i: (i,)),
            pl.BlockSpec((RB,), lambda i: (i,)),
        ],
        out_shape=[
            jax.ShapeDtypeStruct((NP, D), jnp.float32),
            jax.ShapeDtypeStruct((NP,), jnp.float32),
            jax.ShapeDtypeStruct((NP,), jnp.float32),
        ],
    )(x, wk, wb, w1, b1, w2, b2, g, b)


def _combine(parts, s_parts):
    num = parts[0] + parts[1]
    den = jnp.sum(s_parts, axis=0)[:, None]
    pos = den > 0.0
    return jnp.where(pos, num / jnp.where(pos, den, 1.0), 0.0)


def _dense2_body(p_ref, ps_ref, w1_ref, b1_ref, g_ref, b_ref, e_ref, s1_ref):
    out1 = _combine(p_ref[...], ps_ref[...])
    e = _ln(out1, g_ref[...], b_ref[...])
    e_ref[...] = e
    qw = jnp.dot(e, w1_ref[...], preferred_element_type=jnp.float32) + b1_ref[...]
    s1_ref[...] = jnp.tanh(jnp.sum(qw * e, axis=1))


def _dense2(parts, s_parts, w1, b1, g, b):
    return pl.pallas_call(
        _dense2_body,
        grid=(NP // RB,),
        in_specs=[
            pl.BlockSpec((2, RB, D), lambda i: (0, i, 0)),
            pl.BlockSpec((NW, RB), lambda i: (0, i)),
            pl.BlockSpec((D, D), lambda i: (0, 0)),
            pl.BlockSpec((1, D), lambda i: (0, 0)),
            pl.BlockSpec((1, D), lambda i: (0, 0)),
            pl.BlockSpec((1, D), lambda i: (0, 0)),
        ],
        out_specs=[
            pl.BlockSpec((RB, D), lambda i: (i, 0)),
            pl.BlockSpec((RB,), lambda i: (i,)),
        ],
        out_shape=[
            jax.ShapeDtypeStruct((NP, D), jnp.float32),
            jax.ShapeDtypeStruct((NP,), jnp.float32),
        ],
    )(parts, s_parts, w1, b1, g, b)


def _dense_rel_body(x_ref, wk_ref, wb_ref, w2_ref, b2_ref, g_ref, b_ref,
                    d_ref, s2_ref):
    x = x_ref[...]
    inp = _ln(x, g_ref[...], b_ref[...])
    d = jnp.dot(inp, wk_ref[...], preferred_element_type=jnp.float32) + wb_ref[...]
    rowid = lax.broadcasted_iota(jnp.int32, (RP, 1), 0)
    d_ref[...] = jnp.where(rowid < R, d, 0.0)
    qw = jnp.dot(inp, w2_ref[...], preferred_element_type=jnp.float32) + b2_ref[...]
    s2_ref[...] = jnp.tanh(jnp.sum(qw * inp, axis=1))


def _dense_rel(x, wk, wb, w2, b2, g, b):
    return pl.pallas_call(
        _dense_rel_body,
        grid=(1,),
        in_specs=[
            pl.BlockSpec((RP, D), lambda i: (0, 0)),
            pl.BlockSpec((D, D), lambda i: (0, 0)),
            pl.BlockSpec((1, D), lambda i: (0, 0)),
            pl.BlockSpec((D, D), lambda i: (0, 0)),
            pl.BlockSpec((1, D), lambda i: (0, 0)),
            pl.BlockSpec((1, D), lambda i: (0, 0)),
            pl.BlockSpec((1, D), lambda i: (0, 0)),
        ],
        out_specs=[
            pl.BlockSpec((RP, D), lambda i: (0, 0)),
            pl.BlockSpec((RP,), lambda i: (0,)),
        ],
        out_shape=[
            jax.ShapeDtypeStruct((RP, D), jnp.float32),
            jax.ShapeDtypeStruct((RP,), jnp.float32),
        ],
    )(x, wk, wb, w2, b2, g, b)


def _dense3_body(p_ref, ps_ref, e_ref, g_ref, b_ref, dk_ref, db_ref, res_ref):
    out2 = _combine(p_ref[...], ps_ref[...])
    er = _ln(out2, g_ref[...], b_ref[...])
    res_ref[...] = e_ref[...] + 0.6 * (
        jnp.dot(er, dk_ref[...], preferred_element_type=jnp.float32) + db_ref[...])


def _dense3(parts, s_parts, e_attn, g, b, dk, db):
    return pl.pallas_call(
        _dense3_body,
        grid=(NP // RB,),
        in_specs=[
            pl.BlockSpec((2, RB, D), lambda i: (0, i, 0)),
            pl.BlockSpec((NW, RB), lambda i: (0, i)),
            pl.BlockSpec((RB, D), lambda i: (i, 0)),
            pl.BlockSpec((1, D), lambda i: (0, 0)),
            pl.BlockSpec((1, D), lambda i: (0, 0)),
            pl.BlockSpec((D, D), lambda i: (0, 0)),
            pl.BlockSpec((1, D), lambda i: (0, 0)),
        ],
        out_specs=pl.BlockSpec((RB, D), lambda i: (i, 0)),
        out_shape=jax.ShapeDtypeStruct((NP, D), jnp.float32),
    )(parts, s_parts, e_attn, g, b, dk, db)


def _mesh():
    return plsc.VectorSubcoreMesh(core_axis_name="c", subcore_axis_name="s",
                                  num_cores=NC, num_subcores=NS)


@functools.lru_cache(maxsize=None)
def _make_weights(tbl_rows, nchunks):
    @functools.partial(
        pl.kernel,
        out_type=[
            jax.ShapeDtypeStruct((NW, nchunks, K), jnp.float32),
            jax.ShapeDtypeStruct((NW, NP), jnp.float32),
        ],
        mesh=_mesh(),
        scratch_types=[
            pltpu.VMEM((nchunks, K), jnp.int32),
            pltpu.VMEM((nchunks, K), jnp.int32),
            pltpu.VMEM((NP,), jnp.float32),
            pltpu.VMEM((tbl_rows,), jnp.float32),
            pltpu.VMEM((nchunks, K), jnp.float32),
            pltpu.VMEM((NP,), jnp.float32),
        ],
        compiler_params=pltpu.CompilerParams(needs_layout_passes=False),
    )
    def weights_kernel(er_hbm, ec_hbm, s1_hbm, s2_hbm, w_hbm, outs_hbm,
                       rv, cv, s1v, s2v, wv, sv):
        cid = lax.axis_index("c")
        sid = lax.axis_index("s")
        wid = sid * NC + cid

        pltpu.sync_copy(er_hbm.at[wid], rv)
        pltpu.sync_copy(ec_hbm.at[wid], cv)
        pltpu.sync_copy(s1_hbm, s1v)
        pltpu.sync_copy(s2_hbm, s2v)

        @pl.loop(0, NP // L)
        def _zero_s(i):
            sv[pl.ds(i * L, L)] = jnp.zeros((L,), jnp.float32)

        lane = lax.iota(jnp.int32, L)
        dummy = jnp.int32(NP - 8)

        @pl.loop(0, nchunks)
        def _chunk(c):
            for i in range(K // L):
                r16 = rv[c, pl.ds(i * L, L)]
                c16 = cv[c, pl.ds(i * L, L)]
                ev = plsc.load_gather(s1v, [r16]) + plsc.load_gather(s2v, [c16])
                ev = jnp.maximum(ev, 0.2 * ev)
                w16 = jnp.exp(ev)
                wv[c, pl.ds(i * L, L)] = w16
                rs, wsr = plsc.sort_key_val(r16, w16)
                prev = rs.at[jnp.maximum(lane - 1, 0)].get(
                    mode="promise_in_bounds")
                nxt = rs.at[jnp.minimum(lane + 1, L - 1)].get(
                    mode="promise_in_bounds")
                first = (rs != prev) | (lane == 0)
                last = (rs != nxt) | (lane == L - 1)
                prefix = plsc.cumsum(wsr)
                start = plsc.cummax(jnp.where(first, lane, 0))
                pb = prefix.at[jnp.maximum(start - 1, 0)].get(
                    mode="promise_in_bounds")
                tot = prefix - jnp.where(start > 0, pb, 0.0)
                sidx = jnp.where(last, rs, dummy)
                plsc.addupdate_scatter(sv, [sidx], tot)

        pltpu.sync_copy(wv, w_hbm.at[wid])
        pltpu.sync_copy(sv, outs_hbm.at[wid])

    return weights_kernel


@functools.lru_cache(maxsize=None)
def _make_numer(tbl_rows, nchunks):
    rows_pt = NP // NS

    @functools.partial(
        pl.kernel,
        out_type=jax.ShapeDtypeStruct((NC, NP, D), jnp.float32),
        mesh=_mesh(),
        scratch_types=[
            pltpu.VMEM((nchunks, K), jnp.int32),
            pltpu.VMEM((K,), jnp.int32),
            pltpu.VMEM((K,), jnp.int32),
            pltpu.VMEM((K,), jnp.float32),
            pltpu.VMEM((K,), jnp.float32),
            pltpu.VMEM((K, D), jnp.float32),
            pltpu.VMEM((K, D), jnp.float32),
            pltpu.VMEM_SHARED((NP, D), jnp.float32),
            pltpu.SemaphoreType.DMA,
            pltpu.SemaphoreType.DMA,
        ],
        compiler_params=pltpu.CompilerParams(needs_layout_passes=False),
    )
    def numer_kernel(er_hbm, ec_hbm, w_hbm, tbl_hbm, out_hbm,
                     cv, rbufa, rbufb, wbufa, wbufb, gbufa, gbufb,
                     acc, sema, semb):
        cid = lax.axis_index("c")
        sid = lax.axis_index("s")
        wid = sid * NC + cid

        pltpu.sync_copy(ec_hbm.at[wid], cv)
        tbl = tbl_hbm

        @pl.loop(0, K)
        def _zero_g(e):
            for j in range(D // L):
                gbufa[e, pl.ds(j * L, L)] = jnp.zeros((L,), jnp.float32)

        for z in range(rows_pt // K):
            pltpu.sync_copy(gbufa, acc.at[pl.ds(sid * rows_pt + z * K, K)])
        plsc.subcore_barrier()

        def issue(c, rb, wb, gb, sem):
            base = (wid * nchunks + c) * K
            pltpu.async_copy(er_hbm.at[pl.ds(base, K)], rb, sem)
            pltpu.async_copy(w_hbm.at[pl.ds(base, K)], wb, sem)
            pltpu.async_copy(tbl.at[cv.at[c]], gb, sem)

        def wait(c, rb, wb, gb, sem):
            base = (wid * nchunks + c) * K
            pltpu.make_async_copy(er_hbm.at[pl.ds(base, K)], rb, sem).wait()
            pltpu.make_async_copy(w_hbm.at[pl.ds(base, K)], wb, sem).wait()
            pltpu.make_async_copy(tbl.at[cv.at[c]], gb, sem).wait()

        def process(rb, wb, gb):
            @pl.loop(0, K, unroll=8)
            def _scale(e):
                ws = plsc.load_gather(wb, [jnp.full((L,), e, jnp.int32)])
                for j in range(D // L):
                    gb[e, pl.ds(j * L, L)] = gb[e, pl.ds(j * L, L)] * ws

            pltpu.sync_copy(gb, acc.at[rb], add=True)

        assert nchunks % 2 == 1
        issue(0, rbufa, wbufa, gbufa, sema)

        @pl.loop(0, (nchunks - 1) // 2)
        def _pair(t):
            ca = 2 * t
            cb = 2 * t + 1
            issue(cb, rbufb, wbufb, gbufb, semb)
            wait(ca, rbufa, wbufa, gbufa, sema)
            process(rbufa, wbufa, gbufa)
            issue(ca + 2, rbufa, wbufa, gbufa, sema)
            wait(cb, rbufb, wbufb, gbufb, semb)
            process(rbufb, wbufb, gbufb)

        wait(nchunks - 1, rbufa, wbufa, gbufa, sema)
        process(rbufa, wbufa, gbufa)

        plsc.subcore_barrier()
        pltpu.sync_copy(acc.at[pl.ds(sid * rows_pt, rows_pt)],
                        out_hbm.at[cid, pl.ds(sid * rows_pt, rows_pt)])

    return numer_kernel


def _sparse(rr, cc, tbl, s1, s2, tbl_rows, nchunks):
    w, sparts = _make_weights(tbl_rows, nchunks)(rr, cc, s1, s2)
    parts = _make_numer(tbl_rows, nchunks)(rr.reshape(-1), cc,
                                           w.reshape(-1), tbl)
    return parts, sparts


def kernel(ent_em, rel_em, adj1_index, adj2_rows, adj2_cols, params):
    p = params
    f32 = jnp.float32

    x = jnp.pad(ent_em, ((0, NP - N), (0, 0)))
    xr = jnp.pad(rel_em, ((0, RP - R), (0, 0)))

    row2 = lambda a: a.reshape(1, D)

    value, s1, s2 = _dense1(
        x, p["la_w_k"], row2(p["la_w_b"]),
        p["la_w1_k"], row2(p["la_w1_b"]), p["la_w2_k"], row2(p["la_w2_b"]),
        row2(p["la_ln1_g"]), row2(p["la_ln1_b"]))

    r1 = adj1_index[0].reshape(NW, CH1, K)
    c1 = adj1_index[1].reshape(NW, CH1, K)
    parts1, sparts1 = _sparse(r1, c1, value, s1, s2, NP, CH1)

    e_attn, t1 = _dense2(
        parts1, sparts1, p["er_w1_k"], row2(p["er_w1_b"]),
        row2(p["la_ln2_g"]), row2(p["la_ln2_b"]))

    dtab, t2 = _dense_rel(
        xr, p["er_w_k"], row2(p["er_w_b"]), p["er_w2_k"], row2(p["er_w2_b"]),
        row2(p["er_ln1_g"]), row2(p["er_ln1_b"]))

    pad_r = jnp.full((E2P - E2,), N + 1, jnp.int32)
    pad_c = jnp.full((E2P - E2,), R, jnp.int32)
    r2 = jnp.concatenate([adj2_rows, pad_r]).reshape(NW, CH2, K)
    c2 = jnp.concatenate([adj2_cols, pad_c]).reshape(NW, CH2, K)
    parts2, sparts2 = _sparse(r2, c2, dtab, t1, t2, RP, CH2)

    res = _dense3(
        parts2, sparts2, e_attn, row2(p["er_ln2_g"]), row2(p["er_ln2_b"]),
        p["d1_k"], row2(p["d1_b"]))
    return res[:N]
